# baseline (speedup 1.0000x reference)
import functools

import jax
import jax.numpy as jnp
from jax import lax
from jax.experimental import pallas as pl
from jax.experimental.pallas import tpu as pltpu
from jax.experimental.pallas import tpu_sc as plsc

_B, _C, _H, _W = 4, 96, 384, 384
_N = 32
_L = 16
_NL = _N * _L
_NC, _NS = 2, 16
_NW = _NC * _NS
_SH = _NW // _B
_ROWS = _H // _SH
_HR = _ROWS // 2
_GPR = _W // _L
_CS = 72
_CT = _C - _CS
_HB = 16
_NT = _CS // 3


def _sc_segment_sums(feat, idx):
    mesh = plsc.VectorSubcoreMesh(core_axis_name="c", subcore_axis_name="s")

    @functools.partial(
        pl.kernel,
        out_type=(
            jax.ShapeDtypeStruct((_NW, _CS * _NL), jnp.float32),
            jax.ShapeDtypeStruct((_NW, _NL), jnp.float32),
        ),
        mesh=mesh,
        scratch_types=[
            pltpu.VMEM((_ROWS, _W), jnp.int32),
            pltpu.VMEM((_HR, _W), jnp.float32),
            pltpu.VMEM((_HR, _W), jnp.float32),
            pltpu.VMEM((_HR, _W), jnp.float32),
            pltpu.VMEM((_HR, _W), jnp.float32),
            pltpu.VMEM((_HR, _W), jnp.float32),
            pltpu.VMEM((_HR, _W), jnp.float32),
            pltpu.VMEM((_CS * _NL,), jnp.float32),
            pltpu.VMEM((_NL,), jnp.float32),
            pltpu.SemaphoreType.DMA,
            pltpu.SemaphoreType.DMA,
            pltpu.SemaphoreType.DMA,
        ],
        compiler_params=pltpu.CompilerParams(needs_layout_passes=False),
    )
    def seg_kernel(feat_hbm, idx_hbm, psum_hbm, pcnt_hbm,
                   ivf, f00, f01, f02, f10, f11, f12,
                   acc, cacc, sem0, sem1, isem):
        wid = lax.axis_index("s") * _NC + lax.axis_index("c")
        b = wid // _SH
        sh = wid % _SH
        r0 = sh * _ROWS
        fslots = ((f00, f01, f02), (f10, f11, f12))
        sems = (sem0, sem1)

        pltpu.async_copy(idx_hbm.at[b, pl.ds(r0, _ROWS), :], ivf, isem).wait()
        for q in (0, 1):
            for j in range(3):
                pltpu.async_copy(
                    feat_hbm.at[b, j, pl.ds(r0 + q * _HR, _HR), :],
                    fslots[q][j], sems[q],
                )

        lanes = lax.iota(jnp.int32, _L)
        zeros = jnp.zeros((_L,), jnp.float32)
        ones = jnp.ones((_L,), jnp.float32)

        def _zero_row(r, carry):
            @plsc.parallel_loop(0, _NL // _L, unroll=8)
            def _z(u):
                acc[pl.ds(r * _NL + u * _L, _L)] = zeros
            return carry
        lax.fori_loop(0, _CS, _zero_row, 0)

        @plsc.parallel_loop(0, _NL // _L, unroll=8)
        def _zero_cacc(u):
            cacc[pl.ds(u * _L, _L)] = zeros

        def _prep_row(r, carry):
            @plsc.parallel_loop(0, _GPR, unroll=8)
            def _grp(u):
                iv = ivf[r, pl.ds(u * _L, _L)]
                x = iv * _L + lanes
                ivf[r, pl.ds(u * _L, _L)] = x
                plsc.addupdate_scatter(cacc, [x], ones)
            return carry
        lax.fori_loop(0, _ROWS, _prep_row, 0)

        def _triad(kk, carry):
            ch0 = 3 * kk
            for q in range(2):
                fbs = fslots[q]
                sem = sems[q]
                for j in range(3):
                    pltpu.make_async_copy(
                        feat_hbm.at[b, ch0 + j, pl.ds(r0 + q * _HR, _HR), :],
                        fbs[j], sem,
                    ).wait()
                accs = tuple(
                    acc.at[pl.ds((ch0 + j) * _NL, _NL)] for j in range(3)
                )

                def _row(r, inner, q=q, fbs=fbs, accs=accs):
                    @plsc.parallel_loop(0, _GPR, unroll=8)
                    def _grp(u):
                        x = ivf[r + q * _HR, pl.ds(u * _L, _L)]
                        for j in range(3):
                            v = fbs[j][r, pl.ds(u * _L, _L)]
                            plsc.addupdate_scatter(accs[j], [x], v)
                    return inner
                lax.fori_loop(0, _HR, _row, 0)

                @pl.when(kk + 1 < _NT)
                def _prefetch(q=q, fbs=fbs, sem=sem, ch0=ch0):
                    for j in range(3):
                        pltpu.async_copy(
                            feat_hbm.at[
                                b, ch0 + 3 + j, pl.ds(r0 + q * _HR, _HR), :
                            ],
                            fbs[j], sem,
                        )
            return carry
        lax.fori_loop(0, _NT, _triad, 0)

        pltpu.sync_copy(acc, psum_hbm.at[wid])
        pltpu.sync_copy(cacc, pcnt_hbm.at[wid])

    return seg_kernel(feat, idx)


def _tc_segment_sums(feat, idx):
    def body(f_ref, i_ref, out_ref):
        acc = jnp.zeros((_N, _CT), jnp.float32)
        seg = lax.broadcasted_iota(jnp.int32, (_N, _W), 0)
        for hr in range(_HB):
            xr = f_ref[0, :, hr, :]
            ir = i_ref[0, hr, :]
            oh = (jnp.broadcast_to(ir[None, :], (_N, _W)) == seg)
            acc = acc + lax.dot_general(
                oh.astype(jnp.float32), xr,
                (((1,), (1,)), ((), ())))

        @pl.when(pl.program_id(1) == 0)
        def _init():
            out_ref[0] = acc

        @pl.when(pl.program_id(1) != 0)
        def _accum():
            out_ref[0] += acc

    return pl.pallas_call(
        body,
        grid=(_B, _H // _HB),
        in_specs=[
            pl.BlockSpec((1, _CT, _HB, _W), lambda b, h: (b, _CS // _CT, h, 0)),
            pl.BlockSpec((1, _HB, _W), lambda b, h: (b, h, 0)),
        ],
        out_specs=pl.BlockSpec((1, _N, _CT), lambda b, h: (b, 0, 0)),
        out_shape=jax.ShapeDtypeStruct((_B, _N, _CT), jnp.float32),
    )(feat, idx)


def _tc_finalize(psum, pcnt, tcsum):
    def body(ps_ref, pc_ref, tc_ref, sp_ref, sim_ref):
        x = jnp.sum(ps_ref[0], axis=0)
        cn = jnp.sum(pc_ref[0], axis=0)
        col = lax.broadcasted_iota(jnp.int32, (_NL, _N), 0) // _L
        seg = lax.broadcasted_iota(jnp.int32, (_NL, _N), 1)
        onehot = (col == seg).astype(jnp.float32)
        dims = (((0,), (1,)), ((), ()))
        sums_t = lax.dot_general(onehot, x, dims,
                                 precision=lax.Precision.HIGHEST)
        cnt_t = lax.dot_general(onehot, cn[None, :], dims,
                                precision=lax.Precision.HIGHEST)
        inv = 1.0 / jnp.maximum(cnt_t, 1.0)
        m_sc = sums_t * inv
        m_tc = tc_ref[0] * inv
        sp_ref[0, :, 0:_CS] = m_sc
        sp_ref[0, :, _CS:_C] = m_tc
        gdims = (((1,), (1,)), ((), ()))
        gram = (lax.dot_general(m_sc, m_sc, gdims,
                                precision=lax.Precision.HIGHEST)
                + lax.dot_general(m_tc, m_tc, gdims,
                                  precision=lax.Precision.HIGHEST))
        r = jnp.sum(m_sc * m_sc, axis=1) + jnp.sum(m_tc * m_tc, axis=1)
        sim_ref[0] = 1.0 - 0.5 * (r[:, None] + r[None, :]) + gram

    return pl.pallas_call(
        body,
        grid=(_B,),
        in_specs=[
            pl.BlockSpec((1, _SH, _CS, _NL), lambda b: (b, 0, 0, 0)),
            pl.BlockSpec((1, _SH, _NL), lambda b: (b, 0, 0)),
            pl.BlockSpec((1, _N, _CT), lambda b: (b, 0, 0)),
        ],
        out_specs=[
            pl.BlockSpec((1, _N, _C), lambda b: (b, 0, 0)),
            pl.BlockSpec((1, _N, _N), lambda b: (b, 0, 0)),
        ],
        out_shape=[
            jax.ShapeDtypeStruct((_B, _N, _C), jnp.float32),
            jax.ShapeDtypeStruct((_B, _N, _N), jnp.float32),
        ],
    )(psum, pcnt, tcsum)


def kernel(features, superpixel_indices):
    psum, pcnt = _sc_segment_sums(features, superpixel_indices)
    tcsum = _tc_segment_sums(features, superpixel_indices)
    ps = psum.reshape(_B, _SH, _CS, _NL)
    pc = pcnt.reshape(_B, _SH, _NL)
    sp, sim = _tc_finalize(ps, pc, tcsum)
    return (sp, sim)


# speedup vs baseline: 1.0085x; 1.0085x over previous
import functools

import jax
import jax.numpy as jnp
from jax import lax
from jax.experimental import pallas as pl
from jax.experimental.pallas import tpu as pltpu
from jax.experimental.pallas import tpu_sc as plsc

_B, _C, _H, _W = 4, 96, 384, 384
_N = 32
_L = 16
_NL = _N * _L
_NC, _NS = 2, 16
_NW = _NC * _NS
_SH = _NW // _B
_ROWS = _H // _SH
_HR = _ROWS // 2
_GPR = _W // _L
_CS = 72
_CT = _C - _CS
_HB = 16
_NT = _CS // 3


def _sc_segment_sums(feat, idx):
    mesh = plsc.VectorSubcoreMesh(core_axis_name="c", subcore_axis_name="s")

    @functools.partial(
        pl.kernel,
        out_type=(
            jax.ShapeDtypeStruct((_NW, _CS * _NL), jnp.float32),
            jax.ShapeDtypeStruct((_NW, _NL), jnp.float32),
        ),
        mesh=mesh,
        scratch_types=[
            pltpu.VMEM((_ROWS, _W), jnp.int32),
            pltpu.VMEM((_HR, _W), jnp.float32),
            pltpu.VMEM((_HR, _W), jnp.float32),
            pltpu.VMEM((_HR, _W), jnp.float32),
            pltpu.VMEM((_HR, _W), jnp.float32),
            pltpu.VMEM((_HR, _W), jnp.float32),
            pltpu.VMEM((_HR, _W), jnp.float32),
            pltpu.VMEM((_CS * _NL,), jnp.float32),
            pltpu.VMEM((_NL,), jnp.float32),
            pltpu.SemaphoreType.DMA,
            pltpu.SemaphoreType.DMA,
            pltpu.SemaphoreType.DMA,
        ],
        compiler_params=pltpu.CompilerParams(needs_layout_passes=False),
    )
    def seg_kernel(feat_hbm, idx_hbm, psum_hbm, pcnt_hbm,
                   ivf, f00, f01, f02, f10, f11, f12,
                   acc, cacc, sem0, sem1, isem):
        wid = lax.axis_index("s") * _NC + lax.axis_index("c")
        b = wid // _SH
        sh = wid % _SH
        r0 = sh * _ROWS
        fslots = ((f00, f01, f02), (f10, f11, f12))
        sems = (sem0, sem1)

        pltpu.async_copy(idx_hbm.at[b, pl.ds(r0, _ROWS), :], ivf, isem).wait()
        for q in (0, 1):
            for j in range(3):
                pltpu.async_copy(
                    feat_hbm.at[b, j, pl.ds(r0 + q * _HR, _HR), :],
                    fslots[q][j], sems[q],
                )

        lanes = lax.iota(jnp.int32, _L)
        zeros = jnp.zeros((_L,), jnp.float32)
        ones = jnp.ones((_L,), jnp.float32)

        def _zero_row(r, carry):
            @plsc.parallel_loop(0, _NL // _L, unroll=8)
            def _z(u):
                acc[pl.ds(r * _NL + u * _L, _L)] = zeros
            return carry
        lax.fori_loop(0, _CS, _zero_row, 0)

        @plsc.parallel_loop(0, _NL // _L, unroll=8)
        def _zero_cacc(u):
            cacc[pl.ds(u * _L, _L)] = zeros

        def _prep_row(r, carry):
            @plsc.parallel_loop(0, _GPR, unroll=8)
            def _grp(u):
                iv = ivf[r, pl.ds(u * _L, _L)]
                x = iv * _L + lanes
                ivf[r, pl.ds(u * _L, _L)] = x
                plsc.addupdate_scatter(cacc, [x], ones)
            return carry
        lax.fori_loop(0, _ROWS, _prep_row, 0)

        def _triad(kk, carry):
            ch0 = 3 * kk
            for q in range(2):
                fbs = fslots[q]
                sem = sems[q]
                for j in range(3):
                    pltpu.make_async_copy(
                        feat_hbm.at[b, ch0 + j, pl.ds(r0 + q * _HR, _HR), :],
                        fbs[j], sem,
                    ).wait()
                accs = tuple(
                    acc.at[pl.ds((ch0 + j) * _NL, _NL)] for j in range(3)
                )

                def _row(r, inner, q=q, fbs=fbs, accs=accs):
                    @plsc.parallel_loop(0, _GPR, unroll=4)
                    def _grp(u):
                        x = ivf[r + q * _HR, pl.ds(u * _L, _L)]
                        for j in range(3):
                            v = fbs[j][r, pl.ds(u * _L, _L)]
                            plsc.addupdate_scatter(accs[j], [x], v)
                    return inner
                lax.fori_loop(0, _HR, _row, 0)

                @pl.when(kk + 1 < _NT)
                def _prefetch(q=q, fbs=fbs, sem=sem, ch0=ch0):
                    for j in range(3):
                        pltpu.async_copy(
                            feat_hbm.at[
                                b, ch0 + 3 + j, pl.ds(r0 + q * _HR, _HR), :
                            ],
                            fbs[j], sem,
                        )
            return carry
        lax.fori_loop(0, _NT, _triad, 0)

        pltpu.sync_copy(acc, psum_hbm.at[wid])
        pltpu.sync_copy(cacc, pcnt_hbm.at[wid])

    return seg_kernel(feat, idx)


def _tc_segment_sums(feat, idx):
    def body(f_ref, i_ref, out_ref):
        acc = jnp.zeros((_N, _CT), jnp.float32)
        seg = lax.broadcasted_iota(jnp.int32, (_N, _W), 0)
        for hr in range(_HB):
            xr = f_ref[0, :, hr, :]
            ir = i_ref[0, hr, :]
            oh = (jnp.broadcast_to(ir[None, :], (_N, _W)) == seg)
            acc = acc + lax.dot_general(
                oh.astype(jnp.float32), xr,
                (((1,), (1,)), ((), ())))

        @pl.when(pl.program_id(1) == 0)
        def _init():
            out_ref[0] = acc

        @pl.when(pl.program_id(1) != 0)
        def _accum():
            out_ref[0] += acc

    return pl.pallas_call(
        body,
        grid=(_B, _H // _HB),
        in_specs=[
            pl.BlockSpec((1, _CT, _HB, _W), lambda b, h: (b, _CS // _CT, h, 0)),
            pl.BlockSpec((1, _HB, _W), lambda b, h: (b, h, 0)),
        ],
        out_specs=pl.BlockSpec((1, _N, _CT), lambda b, h: (b, 0, 0)),
        out_shape=jax.ShapeDtypeStruct((_B, _N, _CT), jnp.float32),
    )(feat, idx)


def _tc_finalize(psum, pcnt, tcsum):
    def body(ps_ref, pc_ref, tc_ref, sp_ref, sim_ref):
        x = jnp.sum(ps_ref[0], axis=0)
        cn = jnp.sum(pc_ref[0], axis=0)
        col = lax.broadcasted_iota(jnp.int32, (_NL, _N), 0) // _L
        seg = lax.broadcasted_iota(jnp.int32, (_NL, _N), 1)
        onehot = (col == seg).astype(jnp.float32)
        dims = (((0,), (1,)), ((), ()))
        sums_t = lax.dot_general(onehot, x, dims,
                                 precision=lax.Precision.HIGHEST)
        cnt_t = lax.dot_general(onehot, cn[None, :], dims,
                                precision=lax.Precision.HIGHEST)
        inv = 1.0 / jnp.maximum(cnt_t, 1.0)
        m_sc = sums_t * inv
        m_tc = tc_ref[0] * inv
        sp_ref[0, :, 0:_CS] = m_sc
        sp_ref[0, :, _CS:_C] = m_tc
        gdims = (((1,), (1,)), ((), ()))
        gram = (lax.dot_general(m_sc, m_sc, gdims,
                                precision=lax.Precision.HIGHEST)
                + lax.dot_general(m_tc, m_tc, gdims,
                                  precision=lax.Precision.HIGHEST))
        r = jnp.sum(m_sc * m_sc, axis=1) + jnp.sum(m_tc * m_tc, axis=1)
        sim_ref[0] = 1.0 - 0.5 * (r[:, None] + r[None, :]) + gram

    return pl.pallas_call(
        body,
        grid=(_B,),
        in_specs=[
            pl.BlockSpec((1, _SH, _CS, _NL), lambda b: (b, 0, 0, 0)),
            pl.BlockSpec((1, _SH, _NL), lambda b: (b, 0, 0)),
            pl.BlockSpec((1, _N, _CT), lambda b: (b, 0, 0)),
        ],
        out_specs=[
            pl.BlockSpec((1, _N, _C), lambda b: (b, 0, 0)),
            pl.BlockSpec((1, _N, _N), lambda b: (b, 0, 0)),
        ],
        out_shape=[
            jax.ShapeDtypeStruct((_B, _N, _C), jnp.float32),
            jax.ShapeDtypeStruct((_B, _N, _N), jnp.float32),
        ],
    )(psum, pcnt, tcsum)


def kernel(features, superpixel_indices):
    psum, pcnt = _sc_segment_sums(features, superpixel_indices)
    tcsum = _tc_segment_sums(features, superpixel_indices)
    ps = psum.reshape(_B, _SH, _CS, _NL)
    pc = pcnt.reshape(_B, _SH, _NL)
    sp, sim = _tc_finalize(ps, pc, tcsum)
    return (sp, sim)
